# Initial kernel scaffold; baseline (speedup 1.0000x reference)
import jax
import jax.numpy as jnp
from jax.experimental import pallas as pl


def kernel(net_inst_adj, inst_net_adj_v_drive, inst_net_adj_v_sink, x, phi_w0, phi_b0, phi_w1, phi_b1, psi1_w0, psi1_b0, psi1_w1, psi1_b1, psi2_w0, psi2_b0, psi2_w1, psi2_b1, mlp_w0, mlp_b0, mlp_w1, mlp_b1):
    raise NotImplementedError("write your pallas kernel here")


# speedup vs baseline: 1.2337x; 1.2337x over previous
import jax
import jax.numpy as jnp
from jax.experimental import pallas as pl
from jax.experimental.pallas import tpu as pltpu

N = 4096
D = 256
BM = 512
GRID = N // BM


def _k1(x_ref, a0_ref, pw0_ref, pb0_ref, pw1_ref, pb1_ref, out_ref, h_ref):
    i = pl.program_id(0)

    @pl.when(i == 0)
    def _():
        t = jnp.maximum(
            jnp.dot(x_ref[...], pw0_ref[...], preferred_element_type=jnp.float32)
            + pb0_ref[...],
            0.0,
        )
        h_ref[...] = (
            jnp.dot(t, pw1_ref[...], preferred_element_type=jnp.float32)
            + pb1_ref[...]
        )

    out_ref[...] = jnp.dot(
        a0_ref[...], h_ref[...], preferred_element_type=jnp.float32
    )


def _k2(
    nag_ref, b1_ref, b2_ref, x_ref,
    p1w0_ref, p1b0_ref, p1w1_ref, p1b1_ref,
    p2w0_ref, p2b0_ref, p2w1_ref, p2b1_ref,
    mw0_ref, mb0_ref, mw1_ref, mb1_ref,
    out_ref,
):
    f32 = jnp.float32
    nag = nag_ref[...]
    di = jnp.dot(b1_ref[...], nag, preferred_element_type=f32)
    si = jnp.dot(b2_ref[...], nag, preferred_element_type=f32)

    hd = jnp.maximum(jnp.dot(di, p1w0_ref[...], preferred_element_type=f32)
                     + p1b0_ref[...], 0.0)
    hd = jnp.dot(hd, p1w1_ref[...], preferred_element_type=f32) + p1b1_ref[...]

    hs = jnp.maximum(jnp.dot(si, p2w0_ref[...], preferred_element_type=f32)
                     + p2b0_ref[...], 0.0)
    hs = jnp.dot(hs, p2w1_ref[...], preferred_element_type=f32) + p2b1_ref[...]

    t = (
        jnp.dot(x_ref[...], mw0_ref[0:D, :], preferred_element_type=f32)
        + jnp.dot(hd, mw0_ref[D:2 * D, :], preferred_element_type=f32)
        + jnp.dot(hs, mw0_ref[2 * D:3 * D, :], preferred_element_type=f32)
        + mb0_ref[...]
    )
    t = jnp.maximum(t, 0.0)
    out_ref[...] = (
        jnp.dot(t, mw1_ref[...], preferred_element_type=f32) + mb1_ref[...]
    )


def kernel(net_inst_adj, inst_net_adj_v_drive, inst_net_adj_v_sink, x,
           phi_w0, phi_b0, phi_w1, phi_b1,
           psi1_w0, psi1_b0, psi1_w1, psi1_b1,
           psi2_w0, psi2_b0, psi2_w1, psi2_b1,
           mlp_w0, mlp_b0, mlp_w1, mlp_b1):
    f32 = jnp.float32
    row2 = lambda b: b.reshape(1, -1)

    full = lambda shape: pl.BlockSpec(shape, lambda i: (0, 0))
    rows = lambda w: pl.BlockSpec((BM, w), lambda i: (i, 0))

    net_agg = pl.pallas_call(
        _k1,
        grid=(GRID,),
        in_specs=[
            full((N, D)),
            rows(N),
            full((D, D)), full((1, D)), full((D, D)), full((1, D)),
        ],
        out_specs=rows(D),
        out_shape=jax.ShapeDtypeStruct((N, D), f32),
        scratch_shapes=[pltpu.VMEM((N, D), f32)],
    )(x, net_inst_adj, phi_w0, row2(phi_b0), phi_w1, row2(phi_b1))

    out = pl.pallas_call(
        _k2,
        grid=(GRID,),
        in_specs=[
            full((N, D)),
            rows(N),
            rows(N),
            rows(D),
            full((D, D)), full((1, D)), full((D, D)), full((1, D)),
            full((D, D)), full((1, D)), full((D, D)), full((1, D)),
            full((3 * D, 3 * D)), full((1, 3 * D)),
            full((3 * D, D)), full((1, D)),
        ],
        out_specs=rows(D),
        out_shape=jax.ShapeDtypeStruct((N, D), f32),
    )(net_agg, inst_net_adj_v_drive, inst_net_adj_v_sink, x,
      psi1_w0, row2(psi1_b0), psi1_w1, row2(psi1_b1),
      psi2_w0, row2(psi2_b0), psi2_w1, row2(psi2_b1),
      mlp_w0, row2(mlp_b0), mlp_w1, row2(mlp_b1))
    return out
